# baseline (speedup 1.0000x reference)
import functools

import jax
import jax.numpy as jnp
from jax import lax
from jax.experimental import pallas as pl
from jax.experimental.pallas import tpu as pltpu
from jax.experimental.pallas import tpu_sc as plsc

F32 = jnp.float32
BF16 = jnp.bfloat16

SC_NC = 2
SC_NS = 16
SC_NW = SC_NC * SC_NS
SC_L = 16


def _router_body(x_ref, rw_ref, logits_ref, onehot_ref, score_ref):
    x = x_ref[...].astype(BF16)
    rw = rw_ref[...].astype(BF16)
    l = lax.dot_general(x, rw, (((1,), (1,)), ((), ())),
                        preferred_element_type=F32)
    logits_ref[...] = l
    m = jnp.max(l, axis=1, keepdims=True)
    e_iota = lax.broadcasted_iota(jnp.int32, l.shape, 1)
    first = jnp.min(jnp.where(l >= m, e_iota, l.shape[1]),
                    axis=1, keepdims=True)
    onehot_ref[...] = (e_iota == first).astype(F32)
    score_ref[...] = 1.0 / (1.0 + jnp.exp(-m))


def _router(x, rw, tile):
    T, D = x.shape
    E = rw.shape[0]
    grid = (T // tile,)
    return pl.pallas_call(
        _router_body,
        grid=grid,
        in_specs=[
            pl.BlockSpec((tile, D), lambda i: (i, 0)),
            pl.BlockSpec((E, D), lambda i: (0, 0)),
        ],
        out_specs=[
            pl.BlockSpec((tile, E), lambda i: (i, 0)),
            pl.BlockSpec((tile, E), lambda i: (i, 0)),
            pl.BlockSpec((tile, 1), lambda i: (i, 0)),
        ],
        out_shape=[
            jax.ShapeDtypeStruct((T, E), F32),
            jax.ShapeDtypeStruct((T, E), F32),
            jax.ShapeDtypeStruct((T, 1), F32),
        ],
        compiler_params=pltpu.CompilerParams(
            dimension_semantics=("parallel",)),
    )(x, rw)


def _route_meta_body(nch, ts, nt, oh_ref, dst_ref, meta_ref,
                     rank_s, carry_s, base_s):
    i = pl.program_id(0)
    ch, E = oh_ref.shape

    @pl.when(i == 0)
    def _init():
        carry_s[...] = jnp.zeros_like(carry_s)

    @pl.when(i < nch)
    def _pass1():
        oh = oh_ref[...]
        r_iota = lax.broadcasted_iota(jnp.int32, (ch, ch), 0)
        c_iota = lax.broadcasted_iota(jnp.int32, (ch, ch), 1)
        tri = (c_iota < r_iota).astype(F32)
        rank_mat = lax.dot_general(tri, oh, (((1,), (0,)), ((), ())),
                                   preferred_element_type=F32)
        carry = carry_s[...]
        rank_t = jnp.sum((rank_mat + carry) * oh, axis=1)
        rank_s[pl.ds(i, 1), :] = rank_t.reshape(1, ch)
        carry_s[...] = carry + jnp.sum(oh, axis=0, keepdims=True)

    @pl.when(i == nch)
    def _bases():
        counts = carry_s[...]
        tiles = jnp.floor((counts + float(ts - 1)) / float(ts))
        r8 = lax.broadcasted_iota(jnp.int32, (E, E), 0)
        c8 = lax.broadcasted_iota(jnp.int32, (E, E), 1)
        excl = (r8 < c8).astype(F32)
        starts = lax.dot_general(tiles, excl, (((1,), (0,)), ((), ())),
                                 preferred_element_type=F32)
        base_s[...] = starts * float(ts)
        lane = lax.broadcasted_iota(jnp.int32, (1, 128), 1).astype(F32)
        te = jnp.full((1, 128), 0.0, F32)
        total = jnp.zeros((1, 128), F32)
        for e in range(E):
            s_e = starts[0, e]
            n_e = tiles[0, e]
            te = jnp.where((lane >= s_e) & (n_e > 0.0), float(e), te)
            total = total + n_e
        meta_ref[pl.ds(0, 1), :] = te.astype(jnp.int32)
        meta_ref[pl.ds(1, 1), :] = total.astype(jnp.int32)

    @pl.when(i > nch)
    def _pass2():
        c = i - nch - 1
        oh = oh_ref[...]
        base_tok = jnp.sum(oh * base_s[...], axis=1)
        rank_t = rank_s[pl.ds(c, 1), :].reshape(ch)
        dst_ref[...] = (base_tok + rank_t).astype(jnp.int32)


def _route_meta(onehot, ch, ts, nt):
    T, E = onehot.shape
    nch = T // ch

    def oh_idx(i):
        j = jnp.where(i < nch, i, jnp.maximum(i - nch - 1, 0))
        return (j, 0)

    def dst_idx(i):
        return (jnp.maximum(i - nch - 1, 0),)

    return pl.pallas_call(
        functools.partial(_route_meta_body, nch, ts, nt),
        grid=(2 * nch + 1,),
        in_specs=[pl.BlockSpec((ch, E), oh_idx)],
        out_specs=[
            pl.BlockSpec((ch,), dst_idx),
            pl.BlockSpec((8, 128), lambda i: (0, 0)),
        ],
        out_shape=[
            jax.ShapeDtypeStruct((T,), jnp.int32),
            jax.ShapeDtypeStruct((8, 128), jnp.int32),
        ],
        scratch_shapes=[
            pltpu.VMEM((nch, ch), F32),
            pltpu.VMEM((1, E), F32),
            pltpu.VMEM((1, E), F32),
        ],
        compiler_params=pltpu.CompilerParams(
            dimension_semantics=("arbitrary",)),
    )(onehot)


def _build_src(dst, nslots):
    T = dst.shape[0]
    mesh = plsc.VectorSubcoreMesh(core_axis_name="c", subcore_axis_name="s")

    @functools.partial(
        pl.kernel, mesh=mesh,
        out_type=jax.ShapeDtypeStruct((nslots,), jnp.int32),
        scratch_types=[
            pltpu.VMEM((nslots,), jnp.int32),
            pltpu.VMEM((T,), jnp.int32),
        ],
        compiler_params=pltpu.CompilerParams(needs_layout_passes=False),
    )
    def k(dst_hbm, src_hbm, src_v, dst_v):
        wid = lax.axis_index("s") * SC_NC + lax.axis_index("c")

        @pl.when(wid == 0)
        def _():
            def zinit(j, _):
                vals = lax.rem(j * SC_L + lax.iota(jnp.int32, SC_L), T)
                src_v[pl.ds(j * SC_L, SC_L)] = vals
                return 0
            lax.fori_loop(0, nslots // SC_L, zinit, 0)
            pltpu.sync_copy(dst_hbm, dst_v)

            def scat(j, _):
                iv = dst_v[pl.ds(j * SC_L, SC_L)]
                vals = j * SC_L + lax.iota(jnp.int32, SC_L)
                plsc.store_scatter(src_v, [iv], vals)
                return 0
            lax.fori_loop(0, T // SC_L, scat, 0)
            pltpu.sync_copy(src_v, src_hbm)

    return k(dst)


def _gather_rows(table, idx, chunk):
    V, D = table.shape
    B = idx.shape[0]
    per_w = B // SC_NW
    nchunks = per_w // chunk
    mesh = plsc.VectorSubcoreMesh(core_axis_name="c", subcore_axis_name="s")

    @functools.partial(
        pl.kernel, mesh=mesh,
        out_type=jax.ShapeDtypeStruct((B, D), table.dtype),
        scratch_types=[
            pltpu.VMEM((per_w,), jnp.int32),
            pltpu.VMEM((chunk, D), table.dtype),
            pltpu.VMEM((chunk, D), table.dtype),
            pltpu.SemaphoreType.DMA,
            pltpu.SemaphoreType.DMA,
            pltpu.SemaphoreType.DMA,
            pltpu.SemaphoreType.DMA,
        ],
    )
    def k(table_hbm, idx_hbm, out_hbm, idx_v, buf0, buf1, g0, g1, w0, w1):
        wid = lax.axis_index("s") * SC_NC + lax.axis_index("c")
        base = wid * per_w
        pltpu.sync_copy(idx_hbm.at[pl.ds(base, per_w)], idx_v)
        bufs, gsems, wsems = [buf0, buf1], [g0, g1], [w0, w1]

        def gath(c):
            return pltpu.async_copy(
                table_hbm.at[idx_v.at[pl.ds(c * chunk, chunk)]],
                bufs[c % 2], gsems[c % 2])

        gd = {0: gath(0)}
        wd = {}
        for c in range(nchunks):
            gd[c].wait()
            if c + 1 < nchunks:
                if c >= 1:
                    wd[c - 1].wait()
                gd[c + 1] = gath(c + 1)
            wd[c] = pltpu.async_copy(
                bufs[c % 2], out_hbm.at[pl.ds(base + c * chunk, chunk)],
                wsems[c % 2])
        wd[nchunks - 1].wait()
        if nchunks >= 2:
            wd[nchunks - 2].wait()

    return k(table, idx)


def _group_mlp_body(nf, te_ref, nt_ref, xs_ref, g_ref, u_ref, d_ref, ys_ref):
    t = pl.program_id(0)
    f = pl.program_id(1)

    @pl.when(t < nt_ref[0])
    def _():
        x = xs_ref[...].astype(BF16)
        g = g_ref[0].astype(BF16)
        u = u_ref[0].astype(BF16)
        d = d_ref[0].astype(BF16)
        a = lax.dot_general(x, g, (((1,), (1,)), ((), ())),
                            preferred_element_type=F32)
        b = lax.dot_general(x, u, (((1,), (1,)), ((), ())),
                            preferred_element_type=F32)
        h = ((a / (1.0 + jnp.exp(-a))) * b).astype(BF16)
        contrib = lax.dot_general(h, d, (((1,), (1,)), ((), ())),
                                  preferred_element_type=F32)

        @pl.when(f == 0)
        def _w0():
            ys_ref[...] = contrib

        @pl.when(f > 0)
        def _w1():
            ys_ref[...] += contrib


def _group_mlp(te, nt, xs, eg, eu, ed, ts, ft):
    NS_, D = xs.shape
    E, Fdim, _ = eg.shape
    NT = NS_ // ts
    NF = Fdim // ft
    grid_spec = pltpu.PrefetchScalarGridSpec(
        num_scalar_prefetch=2,
        grid=(NT, NF),
        in_specs=[
            pl.BlockSpec((ts, D), lambda t, f, te_, nt_: (t, 0)),
            pl.BlockSpec((1, ft, D), lambda t, f, te_, nt_: (te_[t], f, 0)),
            pl.BlockSpec((1, ft, D), lambda t, f, te_, nt_: (te_[t], f, 0)),
            pl.BlockSpec((1, D, ft), lambda t, f, te_, nt_: (te_[t], 0, f)),
        ],
        out_specs=pl.BlockSpec((ts, D), lambda t, f, te_, nt_: (t, 0)),
        scratch_shapes=[],
    )
    return pl.pallas_call(
        functools.partial(_group_mlp_body, NF),
        grid_spec=grid_spec,
        out_shape=jax.ShapeDtypeStruct((NS_, D), F32),
        compiler_params=pltpu.CompilerParams(
            dimension_semantics=("parallel", "arbitrary")),
    )(te, nt, xs, eg, eu, ed)


def _shared_body(x_ref, g_ref, u_ref, d_ref, out_ref):
    f = pl.program_id(1)
    x = x_ref[...].astype(BF16)
    g = g_ref[...]
    u = u_ref[...]
    d = d_ref[...]
    a = lax.dot_general(x, g, (((1,), (1,)), ((), ())),
                        preferred_element_type=F32)
    b = lax.dot_general(x, u, (((1,), (1,)), ((), ())),
                        preferred_element_type=F32)
    h = ((a / (1.0 + jnp.exp(-a))) * b).astype(BF16)
    contrib = lax.dot_general(h, d, (((1,), (1,)), ((), ())),
                              preferred_element_type=F32)

    @pl.when(f == 0)
    def _w0():
        out_ref[...] = contrib

    @pl.when(f > 0)
    def _w1():
        out_ref[...] += contrib


def _shared_mlp(x, sg, su, sd, ts, ft):
    T, D = x.shape
    Fdim = sg.shape[0]
    NTT = T // ts
    NF = Fdim // ft
    return pl.pallas_call(
        _shared_body,
        grid=(NTT, NF),
        in_specs=[
            pl.BlockSpec((ts, D), lambda t, f: (t, 0)),
            pl.BlockSpec((ft, D), lambda t, f: (f, 0)),
            pl.BlockSpec((ft, D), lambda t, f: (f, 0)),
            pl.BlockSpec((D, ft), lambda t, f: (0, f)),
        ],
        out_specs=pl.BlockSpec((ts, D), lambda t, f: (t, 0)),
        out_shape=jax.ShapeDtypeStruct((T, D), F32),
        compiler_params=pltpu.CompilerParams(
            dimension_semantics=("parallel", "arbitrary")),
    )(x, sg, su, sd)


def _combine_body(sh_ref, yt_ref, sc_ref, out_ref):
    out_ref[...] = sh_ref[...] + yt_ref[...] * sc_ref[...]


def _combine(sh, ys_tok, score, ts):
    T, D = sh.shape
    return pl.pallas_call(
        _combine_body,
        grid=(T // ts,),
        in_specs=[
            pl.BlockSpec((ts, D), lambda t: (t, 0)),
            pl.BlockSpec((ts, D), lambda t: (t, 0)),
            pl.BlockSpec((ts, 1), lambda t: (t, 0)),
        ],
        out_specs=pl.BlockSpec((ts, D), lambda t: (t, 0)),
        out_shape=jax.ShapeDtypeStruct((T, D), F32),
        compiler_params=pltpu.CompilerParams(
            dimension_semantics=("parallel",)),
    )(sh, ys_tok, score)


def kernel(hidden_states, router_w, shared_gate_w, shared_up_w, shared_down_w,
           expert_gate_w, expert_up_w, expert_down_w):
    x = hidden_states
    T, D = x.shape
    E = router_w.shape[0]
    TS = 1024
    NT = T // TS + E - 1
    NSLOTS = NT * TS

    logits, onehot, score = _router(x, router_w, tile=1024)
    dst, meta = _route_meta(onehot, ch=512, ts=TS, nt=NT)
    te = meta[0, :NT]
    nt = meta[1, :1]
    src = _build_src(dst, NSLOTS)
    xs = _gather_rows(x, src, chunk=24)
    sh = _shared_mlp(x, shared_gate_w.astype(BF16), shared_up_w.astype(BF16),
                     shared_down_w.astype(BF16), ts=512, ft=1024)
    ys = _group_mlp(te, nt, xs, expert_gate_w, expert_up_w, expert_down_w,
                    ts=TS, ft=256)
    ys_tok = _gather_rows(ys, dst, chunk=16)
    out = _combine(sh, ys_tok, score, ts=1024)
    return (out, logits)


# speedup vs baseline: 1.0552x; 1.0552x over previous
import functools

import jax
import jax.numpy as jnp
from jax import lax
from jax.experimental import pallas as pl
from jax.experimental.pallas import tpu as pltpu
from jax.experimental.pallas import tpu_sc as plsc

F32 = jnp.float32
BF16 = jnp.bfloat16

SC_NC = 2
SC_NS = 16
SC_NW = SC_NC * SC_NS
SC_L = 16


def _router_body(x_ref, rw_ref, logits_ref, onehot_ref, score_ref):
    x = x_ref[...].astype(BF16)
    rw = rw_ref[...].astype(BF16)
    l = lax.dot_general(x, rw, (((1,), (1,)), ((), ())),
                        preferred_element_type=F32)
    logits_ref[...] = l
    m = jnp.max(l, axis=1, keepdims=True)
    e_iota = lax.broadcasted_iota(jnp.int32, l.shape, 1)
    first = jnp.min(jnp.where(l >= m, e_iota, l.shape[1]),
                    axis=1, keepdims=True)
    onehot_ref[...] = (e_iota == first).astype(F32)
    score_ref[...] = 1.0 / (1.0 + jnp.exp(-m))


def _router(x, rw, tile):
    T, D = x.shape
    E = rw.shape[0]
    grid = (T // tile,)
    return pl.pallas_call(
        _router_body,
        grid=grid,
        in_specs=[
            pl.BlockSpec((tile, D), lambda i: (i, 0)),
            pl.BlockSpec((E, D), lambda i: (0, 0)),
        ],
        out_specs=[
            pl.BlockSpec((tile, E), lambda i: (i, 0)),
            pl.BlockSpec((tile, E), lambda i: (i, 0)),
            pl.BlockSpec((tile, 1), lambda i: (i, 0)),
        ],
        out_shape=[
            jax.ShapeDtypeStruct((T, E), F32),
            jax.ShapeDtypeStruct((T, E), F32),
            jax.ShapeDtypeStruct((T, 1), F32),
        ],
        compiler_params=pltpu.CompilerParams(
            dimension_semantics=("parallel",)),
    )(x, rw)


def _route_meta_body(nch, ts, nt, oh_ref, dst_ref, meta_ref,
                     rank_s, carry_s, base_s):
    i = pl.program_id(0)
    ch, E = oh_ref.shape

    @pl.when(i == 0)
    def _init():
        carry_s[...] = jnp.zeros_like(carry_s)

    @pl.when(i < nch)
    def _pass1():
        oh = oh_ref[...]
        r_iota = lax.broadcasted_iota(jnp.int32, (ch, ch), 0)
        c_iota = lax.broadcasted_iota(jnp.int32, (ch, ch), 1)
        tri = (c_iota < r_iota).astype(F32)
        rank_mat = lax.dot_general(tri, oh, (((1,), (0,)), ((), ())),
                                   preferred_element_type=F32)
        carry = carry_s[...]
        rank_t = jnp.sum((rank_mat + carry) * oh, axis=1)
        rank_s[pl.ds(i, 1), :] = rank_t.reshape(1, ch)
        carry_s[...] = carry + jnp.sum(oh, axis=0, keepdims=True)

    @pl.when(i == nch)
    def _bases():
        counts = carry_s[...]
        tiles = jnp.floor((counts + float(ts - 1)) / float(ts))
        r8 = lax.broadcasted_iota(jnp.int32, (E, E), 0)
        c8 = lax.broadcasted_iota(jnp.int32, (E, E), 1)
        excl = (r8 < c8).astype(F32)
        starts = lax.dot_general(tiles, excl, (((1,), (0,)), ((), ())),
                                 preferred_element_type=F32)
        base_s[...] = starts * float(ts)
        lane = lax.broadcasted_iota(jnp.int32, (1, 128), 1).astype(F32)
        te = jnp.full((1, 128), 0.0, F32)
        total = jnp.zeros((1, 128), F32)
        for e in range(E):
            s_e = starts[0, e]
            n_e = tiles[0, e]
            te = jnp.where((lane >= s_e) & (n_e > 0.0), float(e), te)
            total = total + n_e
        meta_ref[pl.ds(0, 1), :] = te.astype(jnp.int32)
        meta_ref[pl.ds(1, 1), :] = total.astype(jnp.int32)

    @pl.when(i > nch)
    def _pass2():
        c = i - nch - 1
        oh = oh_ref[...]
        base_tok = jnp.sum(oh * base_s[...], axis=1)
        rank_t = rank_s[pl.ds(c, 1), :].reshape(ch)
        dst_ref[...] = (base_tok + rank_t).astype(jnp.int32)


def _route_meta(onehot, ch, ts, nt):
    T, E = onehot.shape
    nch = T // ch

    def oh_idx(i):
        j = jnp.where(i < nch, i, jnp.maximum(i - nch - 1, 0))
        return (j, 0)

    def dst_idx(i):
        return (jnp.maximum(i - nch - 1, 0),)

    return pl.pallas_call(
        functools.partial(_route_meta_body, nch, ts, nt),
        grid=(2 * nch + 1,),
        in_specs=[pl.BlockSpec((ch, E), oh_idx)],
        out_specs=[
            pl.BlockSpec((ch,), dst_idx),
            pl.BlockSpec((8, 128), lambda i: (0, 0)),
        ],
        out_shape=[
            jax.ShapeDtypeStruct((T,), jnp.int32),
            jax.ShapeDtypeStruct((8, 128), jnp.int32),
        ],
        scratch_shapes=[
            pltpu.VMEM((nch, ch), F32),
            pltpu.VMEM((1, E), F32),
            pltpu.VMEM((1, E), F32),
        ],
        compiler_params=pltpu.CompilerParams(
            dimension_semantics=("arbitrary",)),
    )(onehot)


def _build_src(dst, nslots):
    T = dst.shape[0]
    mesh = plsc.VectorSubcoreMesh(core_axis_name="c", subcore_axis_name="s")

    @functools.partial(
        pl.kernel, mesh=mesh,
        out_type=jax.ShapeDtypeStruct((nslots,), jnp.int32),
        scratch_types=[
            pltpu.VMEM((nslots,), jnp.int32),
            pltpu.VMEM((T,), jnp.int32),
        ],
        compiler_params=pltpu.CompilerParams(needs_layout_passes=False),
    )
    def k(dst_hbm, src_hbm, src_v, dst_v):
        wid = lax.axis_index("s") * SC_NC + lax.axis_index("c")

        @pl.when(wid == 0)
        def _():
            def zinit(j, _):
                vals = lax.rem(j * SC_L + lax.iota(jnp.int32, SC_L), T)
                src_v[pl.ds(j * SC_L, SC_L)] = vals
                return 0
            lax.fori_loop(0, nslots // SC_L, zinit, 0)
            pltpu.sync_copy(dst_hbm, dst_v)

            def scat(j, _):
                iv = dst_v[pl.ds(j * SC_L, SC_L)]
                vals = j * SC_L + lax.iota(jnp.int32, SC_L)
                plsc.store_scatter(src_v, [iv], vals)
                return 0
            lax.fori_loop(0, T // SC_L, scat, 0)
            pltpu.sync_copy(src_v, src_hbm)

    return k(dst)


def _gather_rows(table, idx, chunk):
    V, D = table.shape
    B = idx.shape[0]
    per_w = B // SC_NW
    nchunks = per_w // chunk
    mesh = plsc.VectorSubcoreMesh(core_axis_name="c", subcore_axis_name="s")

    @functools.partial(
        pl.kernel, mesh=mesh,
        out_type=jax.ShapeDtypeStruct((B, D), table.dtype),
        scratch_types=[
            pltpu.VMEM((per_w,), jnp.int32),
            pltpu.VMEM((chunk, D), table.dtype),
            pltpu.VMEM((chunk, D), table.dtype),
            pltpu.SemaphoreType.DMA,
            pltpu.SemaphoreType.DMA,
            pltpu.SemaphoreType.DMA,
            pltpu.SemaphoreType.DMA,
        ],
    )
    def k(table_hbm, idx_hbm, out_hbm, idx_v, buf0, buf1, g0, g1, w0, w1):
        wid = lax.axis_index("s") * SC_NC + lax.axis_index("c")
        base = wid * per_w
        pltpu.sync_copy(idx_hbm.at[pl.ds(base, per_w)], idx_v)
        bufs, gsems, wsems = [buf0, buf1], [g0, g1], [w0, w1]

        def gath(c):
            return pltpu.async_copy(
                table_hbm.at[idx_v.at[pl.ds(c * chunk, chunk)]],
                bufs[c % 2], gsems[c % 2])

        gd = {0: gath(0)}
        wd = {}
        for c in range(nchunks):
            gd[c].wait()
            if c + 1 < nchunks:
                if c >= 1:
                    wd[c - 1].wait()
                gd[c + 1] = gath(c + 1)
            wd[c] = pltpu.async_copy(
                bufs[c % 2], out_hbm.at[pl.ds(base + c * chunk, chunk)],
                wsems[c % 2])
        wd[nchunks - 1].wait()
        if nchunks >= 2:
            wd[nchunks - 2].wait()

    return k(table, idx)


def _group_mlp_body(nf, te_ref, nt_ref, xs_ref, g_ref, u_ref, d_ref, ys_ref):
    t = pl.program_id(0)
    f = pl.program_id(1)

    @pl.when(t < nt_ref[0])
    def _():
        x = xs_ref[...].astype(BF16)
        g = g_ref[0].astype(BF16)
        u = u_ref[0].astype(BF16)
        d = d_ref[0].astype(BF16)
        a = lax.dot_general(x, g, (((1,), (1,)), ((), ())),
                            preferred_element_type=F32)
        b = lax.dot_general(x, u, (((1,), (1,)), ((), ())),
                            preferred_element_type=F32)
        h = ((a / (1.0 + jnp.exp(-a))) * b).astype(BF16)
        contrib = lax.dot_general(h, d, (((1,), (1,)), ((), ())),
                                  preferred_element_type=F32)

        @pl.when(f == 0)
        def _w0():
            ys_ref[...] = contrib

        @pl.when(f > 0)
        def _w1():
            ys_ref[...] += contrib


def _group_mlp(te, nt, xs, eg, eu, ed, ts, ft):
    NS_, D = xs.shape
    E, Fdim, _ = eg.shape
    NT = NS_ // ts
    NF = Fdim // ft
    grid_spec = pltpu.PrefetchScalarGridSpec(
        num_scalar_prefetch=2,
        grid=(NT, NF),
        in_specs=[
            pl.BlockSpec((ts, D), lambda t, f, te_, nt_: (t, 0)),
            pl.BlockSpec((1, ft, D), lambda t, f, te_, nt_: (te_[t], f, 0)),
            pl.BlockSpec((1, ft, D), lambda t, f, te_, nt_: (te_[t], f, 0)),
            pl.BlockSpec((1, D, ft), lambda t, f, te_, nt_: (te_[t], 0, f)),
        ],
        out_specs=pl.BlockSpec((ts, D), lambda t, f, te_, nt_: (t, 0)),
        scratch_shapes=[],
    )
    return pl.pallas_call(
        functools.partial(_group_mlp_body, NF),
        grid_spec=grid_spec,
        out_shape=jax.ShapeDtypeStruct((NS_, D), F32),
        compiler_params=pltpu.CompilerParams(
            dimension_semantics=("parallel", "arbitrary")),
    )(te, nt, xs, eg, eu, ed)


def _shared_body(x_ref, g_ref, u_ref, d_ref, out_ref):
    f = pl.program_id(1)
    x = x_ref[...].astype(BF16)
    g = g_ref[...]
    u = u_ref[...]
    d = d_ref[...]
    a = lax.dot_general(x, g, (((1,), (1,)), ((), ())),
                        preferred_element_type=F32)
    b = lax.dot_general(x, u, (((1,), (1,)), ((), ())),
                        preferred_element_type=F32)
    h = ((a / (1.0 + jnp.exp(-a))) * b).astype(BF16)
    contrib = lax.dot_general(h, d, (((1,), (1,)), ((), ())),
                              preferred_element_type=F32)

    @pl.when(f == 0)
    def _w0():
        out_ref[...] = contrib

    @pl.when(f > 0)
    def _w1():
        out_ref[...] += contrib


def _shared_mlp(x, sg, su, sd, ts, ft):
    T, D = x.shape
    Fdim = sg.shape[0]
    NTT = T // ts
    NF = Fdim // ft
    return pl.pallas_call(
        _shared_body,
        grid=(NTT, NF),
        in_specs=[
            pl.BlockSpec((ts, D), lambda t, f: (t, 0)),
            pl.BlockSpec((ft, D), lambda t, f: (f, 0)),
            pl.BlockSpec((ft, D), lambda t, f: (f, 0)),
            pl.BlockSpec((D, ft), lambda t, f: (0, f)),
        ],
        out_specs=pl.BlockSpec((ts, D), lambda t, f: (t, 0)),
        out_shape=jax.ShapeDtypeStruct((T, D), F32),
        compiler_params=pltpu.CompilerParams(
            dimension_semantics=("parallel", "arbitrary")),
    )(x, sg, su, sd)


def _combine_body(sh_ref, yt_ref, sc_ref, out_ref):
    out_ref[...] = sh_ref[...] + yt_ref[...] * sc_ref[...]


def _combine(sh, ys_tok, score, ts):
    T, D = sh.shape
    return pl.pallas_call(
        _combine_body,
        grid=(T // ts,),
        in_specs=[
            pl.BlockSpec((ts, D), lambda t: (t, 0)),
            pl.BlockSpec((ts, D), lambda t: (t, 0)),
            pl.BlockSpec((ts, 1), lambda t: (t, 0)),
        ],
        out_specs=pl.BlockSpec((ts, D), lambda t: (t, 0)),
        out_shape=jax.ShapeDtypeStruct((T, D), F32),
        compiler_params=pltpu.CompilerParams(
            dimension_semantics=("parallel",)),
    )(sh, ys_tok, score)


def kernel(hidden_states, router_w, shared_gate_w, shared_up_w, shared_down_w,
           expert_gate_w, expert_up_w, expert_down_w):
    x = hidden_states
    T, D = x.shape
    E = router_w.shape[0]
    TS = 512
    NT = T // TS + E
    NSLOTS = NT * TS

    logits, onehot, score = _router(x, router_w, tile=1024)
    dst, meta = _route_meta(onehot, ch=512, ts=TS, nt=NT)
    te = meta[0, :NT]
    nt = meta[1, :1]
    src = _build_src(dst, NSLOTS)
    xs = _gather_rows(x, src, chunk=24)
    sh = _shared_mlp(x, shared_gate_w.astype(BF16), shared_up_w.astype(BF16),
                     shared_down_w.astype(BF16), ts=512, ft=1024)
    ys = _group_mlp(te, nt, xs, expert_gate_w, expert_up_w, expert_down_w,
                    ts=TS, ft=512)
    ys_tok = _gather_rows(ys, dst, chunk=16)
    out = _combine(sh, ys_tok, score, ts=1024)
    return (out, logits)
